# baseline (speedup 1.0000x reference)
import functools

import jax
import jax.numpy as jnp
from jax import lax
from jax.experimental import pallas as pl
from jax.experimental.pallas import tpu as pltpu
from jax.experimental.pallas import tpu_sc as plsc


def _sc_build_weight(C, P, wp_hbm, zo_hbm, zw_hbm, w_hbm, wp_v, zo_v, zw_v, w_v):
    c = lax.axis_index("c")
    s = lax.axis_index("s")

    @pl.when((c == 0) & (s == 0))
    def _():
        pltpu.sync_copy(wp_hbm, wp_v)
        pltpu.sync_copy(zo_hbm, zo_v)
        pltpu.sync_copy(zw_hbm, zw_v)
        lanes = lax.broadcasted_iota(jnp.int32, (16,), 0)
        last = jnp.full((16,), 15, jnp.int32)
        dnums = lax.GatherDimensionNumbers(
            offset_dims=(), collapsed_slice_dims=(0,), start_index_map=(0,))

        def take16(v, idx):
            return lax.gather(
                v, idx[:, None], dnums, (1,),
                mode=lax.GatherScatterMode.PROMISE_IN_BOUNDS)

        tot = jnp.zeros((16,), jnp.int32)
        for j in range(C // 16):
            zw16 = zw_v[pl.ds(j * 16, 16)]
            pc = plsc.cumsum(1 - zw16)
            tot = tot + take16(pc, last)
        ipi = tot // P
        carry = jnp.zeros((16,), jnp.int32)
        for j in range(C // 16):
            zo16 = zo_v[pl.ds(j * 16, 16)]
            zw16 = zw_v[pl.ds(j * 16, 16)]
            rank1 = plsc.cumsum(1 - zw16) + carry
            carry = take16(rank1, last)
            idx = jnp.clip((rank1 - 1) // ipi, 0, P - 1)
            g = plsc.load_gather(wp_v, [idx])
            w16 = jnp.where((zw16 == 0) & (zo16 == 0), g, jnp.float32(0.0))
            w_v[pl.ds(j * 16, 16)] = w16
        pltpu.sync_copy(w_v, w_hbm)


def _build_weight_sc(weight_param, zo32, zw32):
    C = zo32.shape[0]
    P = weight_param.shape[0]
    mesh = plsc.VectorSubcoreMesh(core_axis_name="c", subcore_axis_name="s")
    k = pl.kernel(
        functools.partial(_sc_build_weight, C, P),
        mesh=mesh,
        out_type=jax.ShapeDtypeStruct((C,), jnp.float32),
        scratch_types=[
            pltpu.VMEM((P,), jnp.float32),
            pltpu.VMEM((C,), jnp.int32),
            pltpu.VMEM((C,), jnp.int32),
            pltpu.VMEM((C,), jnp.float32),
        ],
        compiler_params=pltpu.CompilerParams(needs_layout_passes=False),
    )
    return k(weight_param, zo32, zw32)


def _mul_kernel(w_ref, x_ref, o_ref, w8_ref):
    C = x_ref.shape[1]

    @pl.when(pl.program_id(0) == 0)
    def _():
        w8_ref[...] = jnp.broadcast_to(w_ref[...], w8_ref.shape)

    R = x_ref.shape[0]
    xv = x_ref[...].reshape(R // 8, 8, C)
    o_ref[...] = (xv * w8_ref[...][None]).reshape(R, C)


def kernel(x, weight_param, zero_outputs, zero_weights, inputs_per_item):
    B, C = x.shape
    R = 2048
    zo32 = zero_outputs.astype(jnp.int32)
    zw32 = zero_weights.astype(jnp.int32)
    w = _build_weight_sc(weight_param, zo32, zw32).reshape(1, C)
    return pl.pallas_call(
        _mul_kernel,
        grid=(B // R,),
        in_specs=[
            pl.BlockSpec((1, C), lambda i: (0, 0)),
            pl.BlockSpec((R, C), lambda i: (i, 0)),
        ],
        out_specs=pl.BlockSpec((R, C), lambda i: (i, 0)),
        out_shape=jax.ShapeDtypeStruct((B, C), x.dtype),
        scratch_shapes=[pltpu.VMEM((8, C), jnp.float32)],
        compiler_params=pltpu.CompilerParams(
            dimension_semantics=("arbitrary",),
        ),
    )(w, x)
